# baseline (device time: 107073 ns/iter reference)
import jax
import jax.numpy as jnp
from jax import lax
from jax.experimental import pallas as pl
from jax.experimental.pallas import tpu as pltpu

N_DEV = 4


def kernel(x):
    m_per, n = x.shape
    n_per = n // N_DEV
    m_total = N_DEV * m_per

    def body(x_ref, out_ref, stage_ref, stage_sems, send_sems, recv_sems,
             local_sem):
        my_i = lax.axis_index("i")

        stages = []
        for d in range(1, N_DEV):
            dst = lax.rem(my_i + d, N_DEV)
            cp = pltpu.make_async_copy(
                src_ref=x_ref.at[:, pl.ds(dst * n_per, n_per)],
                dst_ref=stage_ref.at[d - 1],
                sem=stage_sems.at[d - 1],
            )
            cp.start()
            stages.append(cp)

        local = pltpu.make_async_copy(
            src_ref=x_ref.at[:, pl.ds(my_i * n_per, n_per)],
            dst_ref=out_ref.at[pl.ds(my_i * m_per, m_per), :],
            sem=local_sem,
        )
        local.start()

        sends = []
        for d in range(1, N_DEV):
            dst = lax.rem(my_i + d, N_DEV)
            stages[d - 1].wait()
            rdma = pltpu.make_async_remote_copy(
                src_ref=stage_ref.at[d - 1],
                dst_ref=out_ref.at[pl.ds(my_i * m_per, m_per), :],
                send_sem=send_sems.at[d - 1],
                recv_sem=recv_sems.at[d - 1],
                device_id=(dst,),
                device_id_type=pl.DeviceIdType.MESH,
            )
            rdma.start()
            sends.append(rdma)

        for rdma in sends:
            rdma.wait_send()
        local.wait()

        for d in range(1, N_DEV):
            src = lax.rem(my_i - d + N_DEV, N_DEV)
            recv = pltpu.make_async_remote_copy(
                src_ref=stage_ref.at[d - 1],
                dst_ref=out_ref.at[pl.ds(src * m_per, m_per), :],
                send_sem=send_sems.at[d - 1],
                recv_sem=recv_sems.at[d - 1],
                device_id=(src,),
                device_id_type=pl.DeviceIdType.MESH,
            )
            recv.wait_recv()

    return pl.pallas_call(
        body,
        out_shape=jax.ShapeDtypeStruct((m_total, n_per), x.dtype),
        in_specs=[pl.BlockSpec(memory_space=pl.ANY)],
        out_specs=pl.BlockSpec(memory_space=pl.ANY),
        scratch_shapes=[
            pltpu.VMEM((N_DEV - 1, m_per, n_per), x.dtype),
            pltpu.SemaphoreType.DMA((N_DEV - 1,)),
            pltpu.SemaphoreType.DMA((N_DEV - 1,)),
            pltpu.SemaphoreType.DMA((N_DEV - 1,)),
            pltpu.SemaphoreType.DMA,
        ],
    )(x)


# device time: 100932 ns/iter; 1.0608x vs baseline; 1.0608x over previous
import jax
import jax.numpy as jnp
from jax import lax
from jax.experimental import pallas as pl
from jax.experimental.pallas import tpu as pltpu

N_DEV = 4


def kernel(x):
    m_per, n = x.shape
    n_per = n // N_DEV
    m_total = N_DEV * m_per

    def body(x_ref, out_ref, send_sems, recv_sems, local_sem):
        my_i = lax.axis_index("i")

        barrier_sem = pltpu.get_barrier_semaphore()
        for d in range(1, N_DEV):
            peer = lax.rem(my_i + d, N_DEV)
            pl.semaphore_signal(
                barrier_sem, inc=1,
                device_id=(peer,), device_id_type=pl.DeviceIdType.MESH,
            )
        pl.semaphore_wait(barrier_sem, N_DEV - 1)

        sends = []
        for d in range(1, N_DEV):
            dst = lax.rem(my_i + d, N_DEV)
            rdma = pltpu.make_async_remote_copy(
                src_ref=x_ref.at[:, pl.ds(dst * n_per, n_per)],
                dst_ref=out_ref.at[pl.ds(my_i * m_per, m_per), :],
                send_sem=send_sems.at[d - 1],
                recv_sem=recv_sems.at[d - 1],
                device_id=(dst,),
                device_id_type=pl.DeviceIdType.MESH,
            )
            rdma.start()
            sends.append(rdma)

        local = pltpu.make_async_copy(
            src_ref=x_ref.at[:, pl.ds(my_i * n_per, n_per)],
            dst_ref=out_ref.at[pl.ds(my_i * m_per, m_per), :],
            sem=local_sem,
        )
        local.start()

        for rdma in sends:
            rdma.wait_send()
        local.wait()

        for d in range(1, N_DEV):
            src = lax.rem(my_i - d + N_DEV, N_DEV)
            recv = pltpu.make_async_remote_copy(
                src_ref=x_ref.at[:, pl.ds(src * n_per, n_per)],
                dst_ref=out_ref.at[pl.ds(src * m_per, m_per), :],
                send_sem=send_sems.at[d - 1],
                recv_sem=recv_sems.at[d - 1],
                device_id=(src,),
                device_id_type=pl.DeviceIdType.MESH,
            )
            recv.wait_recv()

    return pl.pallas_call(
        body,
        out_shape=jax.ShapeDtypeStruct((m_total, n_per), x.dtype),
        in_specs=[pl.BlockSpec(memory_space=pl.ANY)],
        out_specs=pl.BlockSpec(memory_space=pl.ANY),
        scratch_shapes=[
            pltpu.SemaphoreType.DMA((N_DEV - 1,)),
            pltpu.SemaphoreType.DMA((N_DEV - 1,)),
            pltpu.SemaphoreType.DMA,
        ],
        compiler_params=pltpu.CompilerParams(collective_id=0),
    )(x)
